# baseline (speedup 1.0000x reference)
import functools

import jax
import jax.numpy as jnp
from jax import lax
from jax.experimental import pallas as pl
from jax.experimental.pallas import tpu as pltpu
from jax.experimental.pallas import tpu_sc as plsc

VOCAB = 100000
EMBD = 64
HIDDEN = 128
B = 1024
L = 20

NC = 2
NS = 16
NW = NC * NS
BPW = B // NW
IPW = BPW * L
ICH = 128
NCH = IPW // ICH

VT = 2048
NT = (VOCAB + VT - 1) // VT
NTD = NT - 1
LANES = 128
NQ = 4
QR = B // NQ
OBUF = 4


def _sc_gather_mean(idx_flat, emb):
    @functools.partial(
        pl.kernel,
        out_type=jax.ShapeDtypeStruct((B, EMBD), jnp.float32),
        mesh=plsc.VectorSubcoreMesh(
            core_axis_name="c", subcore_axis_name="s",
            num_cores=NC, num_subcores=NS),
        scratch_types=[
            pltpu.VMEM((IPW,), jnp.int32),
            pltpu.VMEM((IPW, EMBD), jnp.float32),
            pltpu.VMEM((BPW, EMBD), jnp.float32),
            pltpu.SemaphoreType.DMA,
        ],
        compiler_params=pltpu.CompilerParams(use_tc_tiling_on_sc=False),
    )
    def k(idx_hbm, emb_hbm, out_hbm, idx_v, rows_v, out_v, sem):
        wid = lax.axis_index("s") * NC + lax.axis_index("c")
        pltpu.sync_copy(idx_hbm.at[pl.ds(wid * IPW, IPW)], idx_v)
        copies = []
        for ch in range(NCH):
            copies.append(pltpu.async_copy(
                emb_hbm.at[idx_v.at[pl.ds(ch * ICH, ICH)]],
                rows_v.at[pl.ds(ch * ICH, ICH)],
                sem))
        for c in copies:
            c.wait()

        inv_l = jnp.float32(1.0 / L)

        def pool_row(r, carry):
            base = r * L

            def add_l(l, acc):
                row = base + l
                return tuple(
                    acc[c] + rows_v[row, pl.ds(c * 16, 16)]
                    for c in range(EMBD // 16))

            acc0 = tuple(jnp.zeros((16,), jnp.float32)
                         for _ in range(EMBD // 16))
            acc = lax.fori_loop(0, L, add_l, acc0)
            for c in range(EMBD // 16):
                out_v[r, pl.ds(c * 16, 16)] = acc[c] * inv_l
            return carry

        lax.fori_loop(0, BPW, pool_row, 0)
        pltpu.sync_copy(out_v, out_hbm.at[pl.ds(wid * BPW, BPW)])

    return k(idx_flat, emb)


def _tile_logits(hid_rows_bf, w2_ref, b2_ref):
    return lax.dot_general(
        hid_rows_bf, w2_ref[...].astype(jnp.bfloat16),
        (((1,), (1,)), ((), ())),
        preferred_element_type=jnp.float32) + b2_ref[...]


def _out_copy(ring, out_hbm, slot, q, jj, sem):
    return pltpu.make_async_copy(
        ring.at[slot],
        out_hbm.at[pl.ds(q * QR, QR), pl.ds(jj * VT, VT)],
        sem)


def _fused_body(pooled_ref, w1_ref, b1_ref, w2_ref, b2_ref,
                hid_ref, shift_ref, out_hbm,
                m_scr, s_scr, ring, sems):
    p = pl.program_id(0)
    j = pl.program_id(1)

    @pl.when((p == 0) & (j == 0))
    def _init():
        hid_ref[...] = lax.dot_general(
            pooled_ref[...], w1_ref[...],
            (((1,), (1,)), ((), ())),
            preferred_element_type=jnp.float32) + b1_ref[...]
        m_scr[...] = jnp.full((B, LANES), -jnp.inf, jnp.float32)
        s_scr[...] = jnp.zeros((B, LANES), jnp.float32)

    @pl.when(p < NQ)
    def _stats():
        rows = pl.ds(p * QR, QR)
        logits = _tile_logits(
            hid_ref[rows, :].astype(jnp.bfloat16), w2_ref, b2_ref)
        col = j * VT + lax.broadcasted_iota(jnp.int32, (1, VT), 1)
        logits = jnp.where(col < VOCAB, logits, -jnp.inf)

        m_old = m_scr[rows, :]
        m_new = m_old
        for g in range(VT // LANES):
            m_new = jnp.maximum(m_new, logits[:, g * LANES:(g + 1) * LANES])
        s = s_scr[rows, :] * jnp.exp(m_old - m_new)
        for g in range(VT // LANES):
            s = s + jnp.exp(logits[:, g * LANES:(g + 1) * LANES] - m_new)
        s_scr[rows, :] = s
        m_scr[rows, :] = m_new

        @pl.when(j == NT - 1)
        def _finish():
            row_max = jnp.max(m_new, axis=1, keepdims=True)
            row_sum = jnp.sum(s * jnp.exp(m_new - row_max),
                              axis=1, keepdims=True)
            shift_ref[rows, :] = row_max + jnp.log(row_sum)

    @pl.when((p >= 1) & (j < NTD))
    def _out():
        q = p - 1
        slot = lax.rem(j, OBUF)

        @pl.when(j >= OBUF)
        def _wait_same_phase():
            _out_copy(ring, out_hbm, slot, q, j - OBUF,
                      sems.at[slot]).wait()

        @pl.when((j < OBUF) & (p >= 2))
        def _wait_prev_phase():
            _out_copy(ring, out_hbm, slot, q - 1, NTD - OBUF + j,
                      sems.at[slot]).wait()

        rows = pl.ds(q * QR, QR)
        logits = _tile_logits(
            hid_ref[rows, :].astype(jnp.bfloat16), w2_ref, b2_ref)
        ring[slot] = logits - shift_ref[rows, :]
        _out_copy(ring, out_hbm, slot, q, j, sems.at[slot]).start()

    @pl.when((p == NQ) & (j == NT - 1))
    def _drain():
        for dj in range(OBUF):
            jj = NTD - OBUF + dj
            _out_copy(ring, out_hbm, jj % OBUF, NQ - 1, jj,
                      sems.at[jj % OBUF]).wait()


def _tail_body(hid_ref, w2_ref, b2_ref, shift_ref, full_ref, out_ref):
    del full_ref
    logits = _tile_logits(hid_ref[...].astype(jnp.bfloat16), w2_ref, b2_ref)
    out_ref[...] = logits - shift_ref[...]


def kernel(inputs, emb, W1, b1, W2, b2):
    idx_flat = inputs.astype(jnp.int32).reshape(B * L)
    pooled = _sc_gather_mean(idx_flat, emb)

    b1r = b1.reshape(1, HIDDEN)
    b2r = b2.reshape(1, VOCAB)

    hid, shift, out_main = pl.pallas_call(
        _fused_body,
        grid=(NQ + 1, NT),
        in_specs=[
            pl.BlockSpec((B, EMBD), lambda p, j: (0, 0)),
            pl.BlockSpec((HIDDEN, EMBD), lambda p, j: (0, 0)),
            pl.BlockSpec((1, HIDDEN), lambda p, j: (0, 0)),
            pl.BlockSpec((VT, HIDDEN), lambda p, j: (j, 0)),
            pl.BlockSpec((1, VT), lambda p, j: (0, j)),
        ],
        out_specs=[
            pl.BlockSpec((B, HIDDEN), lambda p, j: (0, 0)),
            pl.BlockSpec((B, 1), lambda p, j: (0, 0)),
            pl.BlockSpec(memory_space=pl.ANY),
        ],
        out_shape=[
            jax.ShapeDtypeStruct((B, HIDDEN), jnp.float32),
            jax.ShapeDtypeStruct((B, 1), jnp.float32),
            jax.ShapeDtypeStruct((B, VOCAB), jnp.float32),
        ],
        scratch_shapes=[
            pltpu.VMEM((B, LANES), jnp.float32),
            pltpu.VMEM((B, LANES), jnp.float32),
            pltpu.VMEM((OBUF, QR, VT), jnp.float32),
            pltpu.SemaphoreType.DMA((OBUF,)),
        ],
        compiler_params=pltpu.CompilerParams(
            dimension_semantics=("arbitrary", "arbitrary")),
    )(pooled, W1, b1r, W2, b2r)

    out = pl.pallas_call(
        _tail_body,
        grid=(1,),
        in_specs=[
            pl.BlockSpec((B, HIDDEN), lambda i: (0, 0)),
            pl.BlockSpec((VT, HIDDEN), lambda i: (NT - 1, 0)),
            pl.BlockSpec((1, VT), lambda i: (0, NT - 1)),
            pl.BlockSpec((B, 1), lambda i: (0, 0)),
            pl.BlockSpec(memory_space=pl.ANY),
        ],
        out_specs=pl.BlockSpec((B, VT), lambda i: (0, NT - 1)),
        out_shape=jax.ShapeDtypeStruct((B, VOCAB), jnp.float32),
        input_output_aliases={4: 0},
    )(hid, W2, b2r, shift, out_main)

    return out


# speedup vs baseline: 1.1409x; 1.1409x over previous
import functools

import jax
import jax.numpy as jnp
from jax import lax
from jax.experimental import pallas as pl
from jax.experimental.pallas import tpu as pltpu
from jax.experimental.pallas import tpu_sc as plsc

VOCAB = 100000
EMBD = 64
HIDDEN = 128
B = 1024
L = 20

NC = 2
NS = 16
NW = NC * NS
BPW = B // NW
IPW = BPW * L
ICH = 128
NCH = IPW // ICH

VT = 2048
NT = (VOCAB + VT - 1) // VT
NTD = NT - 1
LANES = 128
NQ = 2
QR = B // NQ
OBUF = 4


def _sc_gather_mean(idx_flat, emb):
    @functools.partial(
        pl.kernel,
        out_type=jax.ShapeDtypeStruct((B, EMBD), jnp.float32),
        mesh=plsc.VectorSubcoreMesh(
            core_axis_name="c", subcore_axis_name="s",
            num_cores=NC, num_subcores=NS),
        scratch_types=[
            pltpu.VMEM((IPW,), jnp.int32),
            pltpu.VMEM((IPW, EMBD), jnp.float32),
            pltpu.VMEM((BPW, EMBD), jnp.float32),
            pltpu.SemaphoreType.DMA,
        ],
        compiler_params=pltpu.CompilerParams(use_tc_tiling_on_sc=False),
    )
    def k(idx_hbm, emb_hbm, out_hbm, idx_v, rows_v, out_v, sem):
        wid = lax.axis_index("s") * NC + lax.axis_index("c")
        pltpu.sync_copy(idx_hbm.at[pl.ds(wid * IPW, IPW)], idx_v)
        copies = []
        for ch in range(NCH):
            copies.append(pltpu.async_copy(
                emb_hbm.at[idx_v.at[pl.ds(ch * ICH, ICH)]],
                rows_v.at[pl.ds(ch * ICH, ICH)],
                sem))
        for c in copies:
            c.wait()

        inv_l = jnp.float32(1.0 / L)

        def pool_row(r, carry):
            base = r * L

            def add_l(l, acc):
                row = base + l
                return tuple(
                    acc[c] + rows_v[row, pl.ds(c * 16, 16)]
                    for c in range(EMBD // 16))

            acc0 = tuple(jnp.zeros((16,), jnp.float32)
                         for _ in range(EMBD // 16))
            acc = lax.fori_loop(0, L, add_l, acc0)
            for c in range(EMBD // 16):
                out_v[r, pl.ds(c * 16, 16)] = acc[c] * inv_l
            return carry

        lax.fori_loop(0, BPW, pool_row, 0)
        pltpu.sync_copy(out_v, out_hbm.at[pl.ds(wid * BPW, BPW)])

    return k(idx_flat, emb)


def _tile_logits(hid_rows_bf, w2_ref, b2_ref):
    return lax.dot_general(
        hid_rows_bf, w2_ref[...].astype(jnp.bfloat16),
        (((1,), (1,)), ((), ())),
        preferred_element_type=jnp.float32) + b2_ref[...]


def _out_copy(ring, out_hbm, slot, q, jj, sem):
    return pltpu.make_async_copy(
        ring.at[slot],
        out_hbm.at[pl.ds(q * QR, QR), pl.ds(jj * VT, VT)],
        sem)


def _fused_body(pooled_ref, w1_ref, b1_ref, w2_ref, b2_ref,
                hid_ref, shift_ref, out_hbm,
                m_scr, s_scr, ring, sems):
    p = pl.program_id(0)
    j = pl.program_id(1)

    @pl.when((p == 0) & (j == 0))
    def _init():
        hid_ref[...] = lax.dot_general(
            pooled_ref[...], w1_ref[...],
            (((1,), (1,)), ((), ())),
            preferred_element_type=jnp.float32) + b1_ref[...]
        m_scr[...] = jnp.full((B, LANES), -jnp.inf, jnp.float32)
        s_scr[...] = jnp.zeros((B, LANES), jnp.float32)

    @pl.when(p < NQ)
    def _stats():
        rows = pl.ds(p * QR, QR)
        logits = _tile_logits(
            hid_ref[rows, :].astype(jnp.bfloat16), w2_ref, b2_ref)
        col = j * VT + lax.broadcasted_iota(jnp.int32, (1, VT), 1)
        logits = jnp.where(col < VOCAB, logits, -jnp.inf)

        m_old = m_scr[rows, :]
        m_new = m_old
        for g in range(VT // LANES):
            m_new = jnp.maximum(m_new, logits[:, g * LANES:(g + 1) * LANES])
        s = s_scr[rows, :] * jnp.exp(m_old - m_new)
        for g in range(VT // LANES):
            s = s + jnp.exp(logits[:, g * LANES:(g + 1) * LANES] - m_new)
        s_scr[rows, :] = s
        m_scr[rows, :] = m_new

        @pl.when(j == NT - 1)
        def _finish():
            row_max = jnp.max(m_new, axis=1, keepdims=True)
            row_sum = jnp.sum(s * jnp.exp(m_new - row_max),
                              axis=1, keepdims=True)
            shift_ref[rows, :] = row_max + jnp.log(row_sum)

    @pl.when((p >= 1) & (j < NTD))
    def _out():
        q = p - 1
        slot = lax.rem(j, OBUF)

        @pl.when(j >= OBUF)
        def _wait_same_phase():
            _out_copy(ring, out_hbm, slot, q, j - OBUF,
                      sems.at[slot]).wait()

        @pl.when((j < OBUF) & (p >= 2))
        def _wait_prev_phase():
            _out_copy(ring, out_hbm, slot, q - 1, NTD - OBUF + j,
                      sems.at[slot]).wait()

        rows = pl.ds(q * QR, QR)
        logits = _tile_logits(
            hid_ref[rows, :].astype(jnp.bfloat16), w2_ref, b2_ref)
        ring[slot] = logits - shift_ref[rows, :]
        _out_copy(ring, out_hbm, slot, q, j, sems.at[slot]).start()

    @pl.when((p == NQ) & (j == NT - 1))
    def _drain():
        for dj in range(OBUF):
            jj = NTD - OBUF + dj
            _out_copy(ring, out_hbm, jj % OBUF, NQ - 1, jj,
                      sems.at[jj % OBUF]).wait()


def _tail_body(hid_ref, w2_ref, b2_ref, shift_ref, full_ref, out_ref):
    del full_ref
    logits = _tile_logits(hid_ref[...].astype(jnp.bfloat16), w2_ref, b2_ref)
    out_ref[...] = logits - shift_ref[...]


def kernel(inputs, emb, W1, b1, W2, b2):
    idx_flat = inputs.astype(jnp.int32).reshape(B * L)
    pooled = _sc_gather_mean(idx_flat, emb)

    b1r = b1.reshape(1, HIDDEN)
    b2r = b2.reshape(1, VOCAB)

    hid, shift, out_main = pl.pallas_call(
        _fused_body,
        grid=(NQ + 1, NT),
        in_specs=[
            pl.BlockSpec((B, EMBD), lambda p, j: (0, 0)),
            pl.BlockSpec((HIDDEN, EMBD), lambda p, j: (0, 0)),
            pl.BlockSpec((1, HIDDEN), lambda p, j: (0, 0)),
            pl.BlockSpec((VT, HIDDEN), lambda p, j: (j, 0)),
            pl.BlockSpec((1, VT), lambda p, j: (0, j)),
        ],
        out_specs=[
            pl.BlockSpec((B, HIDDEN), lambda p, j: (0, 0)),
            pl.BlockSpec((B, 1), lambda p, j: (0, 0)),
            pl.BlockSpec(memory_space=pl.ANY),
        ],
        out_shape=[
            jax.ShapeDtypeStruct((B, HIDDEN), jnp.float32),
            jax.ShapeDtypeStruct((B, 1), jnp.float32),
            jax.ShapeDtypeStruct((B, VOCAB), jnp.float32),
        ],
        scratch_shapes=[
            pltpu.VMEM((B, LANES), jnp.float32),
            pltpu.VMEM((B, LANES), jnp.float32),
            pltpu.VMEM((OBUF, QR, VT), jnp.float32),
            pltpu.SemaphoreType.DMA((OBUF,)),
        ],
        compiler_params=pltpu.CompilerParams(
            dimension_semantics=("arbitrary", "arbitrary")),
    )(pooled, W1, b1r, W2, b2r)

    out = pl.pallas_call(
        _tail_body,
        grid=(1,),
        in_specs=[
            pl.BlockSpec((B, HIDDEN), lambda i: (0, 0)),
            pl.BlockSpec((VT, HIDDEN), lambda i: (NT - 1, 0)),
            pl.BlockSpec((1, VT), lambda i: (0, NT - 1)),
            pl.BlockSpec((B, 1), lambda i: (0, 0)),
            pl.BlockSpec(memory_space=pl.ANY),
        ],
        out_specs=pl.BlockSpec((B, VT), lambda i: (0, NT - 1)),
        out_shape=jax.ShapeDtypeStruct((B, VOCAB), jnp.float32),
        input_output_aliases={4: 0},
    )(hid, W2, b2r, shift, out_main)

    return out
